# baseline (speedup 1.0000x reference)
import functools

import jax
import jax.numpy as jnp
from jax import lax
from jax.experimental import pallas as pl
from jax.experimental.pallas import tpu as pltpu
from jax.experimental.pallas import tpu_sc as plsc

VOCAB = 100000
DIM = 32
CTX = 20
BATCH = 4096
CTXDIM = CTX * DIM

NW = 32
BPW = BATCH // NW

VT = 4096
VTC = 12800


def _sc_gather_sum(inputs, table4):
    mesh = plsc.VectorSubcoreMesh(core_axis_name="c", subcore_axis_name="s")

    @functools.partial(
        pl.kernel,
        out_type=jax.ShapeDtypeStruct((NW, CTXDIM), jnp.float32),
        mesh=mesh,
        scratch_types=[
            pltpu.VMEM((BPW, CTX), jnp.int32),
            pltpu.VMEM((2, BPW), jnp.int32),
            pltpu.VMEM((2, BPW), jnp.int32),
            pltpu.VMEM((2, BPW, 4 * DIM), jnp.float32),
            pltpu.VMEM((CTXDIM,), jnp.float32),
            pltpu.SemaphoreType.DMA,
            pltpu.SemaphoreType.DMA,
        ],
        compiler_params=pltpu.CompilerParams(
            use_tc_tiling_on_sc=False, needs_layout_passes=False),
    )
    def k(in_hbm, table_hbm, out_hbm, blk_v, idx_v, off_v, rows_v, part_v,
          sem0, sem1):
        wid = lax.axis_index("s") * 2 + lax.axis_index("c")
        sems = (sem0, sem1)
        pltpu.sync_copy(in_hbm.at[pl.ds(wid * BPW, BPW)], blk_v)
        lanes = lax.iota(jnp.int32, 16)

        def build_and_fire(c):
            buf = c % 2
            cvec = jnp.full((16,), c, jnp.int32)
            for g in range(BPW // 16):
                vals = plsc.load_gather(blk_v, [g * 16 + lanes, cvec])
                idx_v[buf, pl.ds(g * 16, 16)] = lax.shift_right_logical(vals, 2)
                off_v[buf, pl.ds(g * 16, 16)] = (vals & 3) * DIM
            return pltpu.async_copy(
                table_hbm.at[idx_v.at[buf]], rows_v.at[buf], sems[buf])

        cp = build_and_fire(0)
        for c in range(CTX):
            buf = c % 2
            nxt = cp
            if c + 1 < CTX:
                cp = build_and_fire(c + 1)
            nxt.wait()

            def grp_body(g, carry):
                a0, a1 = carry
                qv = off_v[buf, pl.ds(g * 16, 16)]
                rbase = g * 16
                for j in range(16):
                    q = qv[j]
                    rfull = jnp.full((16,), rbase + j, jnp.int32)
                    a0 = a0 + plsc.load_gather(rows_v.at[buf], [rfull, q + lanes])
                    a1 = a1 + plsc.load_gather(rows_v.at[buf], [rfull, q + 16 + lanes])
                return (a0, a1)

            z = jnp.zeros((16,), jnp.float32)
            a0, a1 = lax.fori_loop(0, BPW // 16, grp_body, (z, z))
            part_v[pl.ds(c * DIM, 16)] = a0
            part_v[pl.ds(c * DIM + 16, 16)] = a1
        pltpu.sync_copy(part_v, out_hbm.at[wid])

    return k(inputs, table4)


def _tc_logits(partials, W, b2):
    grid = (pl.cdiv(VOCAB, VT),)

    def body(part_ref, w_ref, b_ref, out_ref, lse_ref, flat_v, m_sc, s_sc):
        i = pl.program_id(0)

        @pl.when(i == 0)
        def _():
            m_sc[0] = -jnp.inf
            s_sc[0] = jnp.float32(0.0)
            flat_v[...] = jnp.sum(part_ref[...], axis=0, keepdims=True)

        logits = lax.dot_general(
            flat_v[...], w_ref[...], (((1,), (1,)), ((), ())),
            preferred_element_type=jnp.float32) + b_ref[...]
        gidx = i * VT + lax.broadcasted_iota(jnp.int32, (1, VT), 1)
        lm = jnp.where(gidx < VOCAB, logits, -jnp.inf)
        m0 = m_sc[0]
        m1 = jnp.maximum(m0, jnp.max(lm))
        s_sc[0] = s_sc[0] * jnp.exp(m0 - m1) + jnp.sum(jnp.exp(lm - m1))
        m_sc[0] = m1
        out_ref[...] = logits

        @pl.when(i == pl.num_programs(0) - 1)
        def _():
            lse_ref[...] = jnp.broadcast_to(m_sc[0] + jnp.log(s_sc[0]), (1, 1))

    return pl.pallas_call(
        body,
        grid=grid,
        in_specs=[
            pl.BlockSpec((NW, CTXDIM), lambda i: (0, 0)),
            pl.BlockSpec((VT, CTXDIM), lambda i: (i, 0)),
            pl.BlockSpec((1, VT), lambda i: (0, i)),
        ],
        out_specs=[
            pl.BlockSpec((1, VT), lambda i: (0, i)),
            pl.BlockSpec((1, 1), lambda i: (0, 0)),
        ],
        out_shape=[
            jax.ShapeDtypeStruct((1, VOCAB), jnp.float32),
            jax.ShapeDtypeStruct((1, 1), jnp.float32),
        ],
        scratch_shapes=[
            pltpu.VMEM((1, CTXDIM), jnp.float32),
            pltpu.SMEM((1,), jnp.float32),
            pltpu.SMEM((1,), jnp.float32),
        ],
    )(partials, W, b2)


def _tc_subtract(logits, lse):
    def body(l_ref, lse_ref, o_ref):
        o_ref[...] = l_ref[...] - lse_ref[0, 0]

    return pl.pallas_call(
        body,
        grid=(pl.cdiv(VOCAB, VTC),),
        in_specs=[
            pl.BlockSpec((1, VTC), lambda i: (0, i)),
            pl.BlockSpec((1, 1), lambda i: (0, 0)),
        ],
        out_specs=pl.BlockSpec((1, VTC), lambda i: (0, i)),
        out_shape=jax.ShapeDtypeStruct((1, VOCAB), jnp.float32),
    )(logits, lse)


def kernel(inputs, emb_table, W, b):
    table4 = lax.optimization_barrier(emb_table.reshape(VOCAB // 4, 4 * DIM))
    partials = _sc_gather_sum(inputs, table4)
    logits, lse = _tc_logits(partials, W, b.reshape(1, VOCAB))
    return _tc_subtract(logits, lse)


# speedup vs baseline: 1.1334x; 1.1334x over previous
import functools

import jax
import jax.numpy as jnp
from jax import lax
from jax.experimental import pallas as pl
from jax.experimental.pallas import tpu as pltpu
from jax.experimental.pallas import tpu_sc as plsc

VOCAB = 100000
DIM = 32
CTX = 20
BATCH = 4096
CTXDIM = CTX * DIM

NW = 32
BPW = BATCH // NW

VT = 4096
VTC = 12800


def _sc_gather_sum(inputs, emb_table):
    mesh = plsc.VectorSubcoreMesh(core_axis_name="c", subcore_axis_name="s")

    @functools.partial(
        pl.kernel,
        out_type=jax.ShapeDtypeStruct((NW, CTXDIM), jnp.float32),
        mesh=mesh,
        scratch_types=[
            pltpu.VMEM((BPW, CTX), jnp.int32),
            pltpu.VMEM((CTX, BPW), jnp.int32),
            pltpu.VMEM((CTX, BPW, DIM), jnp.float32),
            pltpu.VMEM((CTXDIM,), jnp.float32),
            pltpu.SemaphoreType.DMA,
        ],
        compiler_params=pltpu.CompilerParams(
            use_tc_tiling_on_sc=False, needs_layout_passes=False),
    )
    def k(in_hbm, table_hbm, out_hbm, blk_v, idx_v, rows_v, part_v, sem):
        wid = lax.axis_index("s") * 2 + lax.axis_index("c")
        pltpu.sync_copy(in_hbm.at[pl.ds(wid * BPW, BPW)], blk_v)
        lanes = lax.iota(jnp.int32, 16)

        copies = []
        for c in range(CTX):
            cvec = jnp.full((16,), c, jnp.int32)
            for g in range(BPW // 16):
                vals = plsc.load_gather(blk_v, [g * 16 + lanes, cvec])
                idx_v[c, pl.ds(g * 16, 16)] = vals
            copies.append(pltpu.async_copy(
                table_hbm.at[idx_v.at[c]], rows_v.at[c], sem))

        for c in range(CTX):
            copies[c].wait()

            def row_body(i, carry):
                a0, a1 = carry
                return (a0 + rows_v[c, i, pl.ds(0, 16)],
                        a1 + rows_v[c, i, pl.ds(16, 16)])

            z = jnp.zeros((16,), jnp.float32)
            a0, a1 = lax.fori_loop(0, BPW, row_body, (z, z), unroll=8)
            part_v[pl.ds(c * DIM, 16)] = a0
            part_v[pl.ds(c * DIM + 16, 16)] = a1
        pltpu.sync_copy(part_v, out_hbm.at[wid])

    return k(inputs, emb_table)


def _tc_logits(partials, W, b2):
    grid = (pl.cdiv(VOCAB, VT),)

    def body(part_ref, w_ref, b_ref, out_ref, lse_ref, flat_v, m_sc, s_sc):
        i = pl.program_id(0)

        @pl.when(i == 0)
        def _():
            m_sc[0] = -jnp.inf
            s_sc[0] = jnp.float32(0.0)
            flat_v[...] = jnp.sum(part_ref[...], axis=0, keepdims=True)

        logits = lax.dot_general(
            flat_v[...], w_ref[...], (((1,), (1,)), ((), ())),
            preferred_element_type=jnp.float32) + b_ref[...]
        gidx = i * VT + lax.broadcasted_iota(jnp.int32, (1, VT), 1)
        lm = jnp.where(gidx < VOCAB, logits, -jnp.inf)
        m0 = m_sc[0]
        m1 = jnp.maximum(m0, jnp.max(lm))
        s_sc[0] = s_sc[0] * jnp.exp(m0 - m1) + jnp.sum(jnp.exp(lm - m1))
        m_sc[0] = m1
        out_ref[...] = logits

        @pl.when(i == pl.num_programs(0) - 1)
        def _():
            lse_ref[...] = jnp.broadcast_to(m_sc[0] + jnp.log(s_sc[0]), (1, 1))

    return pl.pallas_call(
        body,
        grid=grid,
        in_specs=[
            pl.BlockSpec((NW, CTXDIM), lambda i: (0, 0)),
            pl.BlockSpec((VT, CTXDIM), lambda i: (i, 0)),
            pl.BlockSpec((1, VT), lambda i: (0, i)),
        ],
        out_specs=[
            pl.BlockSpec((1, VT), lambda i: (0, i)),
            pl.BlockSpec((1, 1), lambda i: (0, 0)),
        ],
        out_shape=[
            jax.ShapeDtypeStruct((1, VOCAB), jnp.float32),
            jax.ShapeDtypeStruct((1, 1), jnp.float32),
        ],
        scratch_shapes=[
            pltpu.VMEM((1, CTXDIM), jnp.float32),
            pltpu.SMEM((1,), jnp.float32),
            pltpu.SMEM((1,), jnp.float32),
        ],
    )(partials, W, b2)


def _tc_subtract(logits, lse):
    def body(l_ref, lse_ref, o_ref):
        o_ref[...] = l_ref[...] - lse_ref[0, 0]

    return pl.pallas_call(
        body,
        grid=(pl.cdiv(VOCAB, VTC),),
        in_specs=[
            pl.BlockSpec((1, VTC), lambda i: (0, i)),
            pl.BlockSpec((1, 1), lambda i: (0, 0)),
        ],
        out_specs=pl.BlockSpec((1, VTC), lambda i: (0, i)),
        out_shape=jax.ShapeDtypeStruct((1, VOCAB), jnp.float32),
    )(logits, lse)


def kernel(inputs, emb_table, W, b):
    partials = _sc_gather_sum(inputs, emb_table)
    logits, lse = _tc_logits(partials, W, b.reshape(1, VOCAB))
    return _tc_subtract(logits, lse)


# speedup vs baseline: 1.1348x; 1.0012x over previous
import functools

import jax
import jax.numpy as jnp
from jax import lax
from jax.experimental import pallas as pl
from jax.experimental.pallas import tpu as pltpu
from jax.experimental.pallas import tpu_sc as plsc

VOCAB = 100000
DIM = 32
CTX = 20
BATCH = 4096
CTXDIM = CTX * DIM

NW = 32
BPW = BATCH // NW

VT = 5120
VTC = 12800


def _sc_gather_sum(inputs, emb_table):
    mesh = plsc.VectorSubcoreMesh(core_axis_name="c", subcore_axis_name="s")

    @functools.partial(
        pl.kernel,
        out_type=jax.ShapeDtypeStruct((NW, CTXDIM), jnp.float32),
        mesh=mesh,
        scratch_types=[
            pltpu.VMEM((BPW, CTX), jnp.int32),
            pltpu.VMEM((CTX, BPW), jnp.int32),
            pltpu.VMEM((CTX, BPW, DIM), jnp.float32),
            pltpu.VMEM((CTXDIM,), jnp.float32),
            pltpu.SemaphoreType.DMA,
        ],
        compiler_params=pltpu.CompilerParams(
            use_tc_tiling_on_sc=False, needs_layout_passes=False),
    )
    def k(in_hbm, table_hbm, out_hbm, blk_v, idx_v, rows_v, part_v, sem):
        wid = lax.axis_index("s") * 2 + lax.axis_index("c")
        pltpu.sync_copy(in_hbm.at[pl.ds(wid * BPW, BPW)], blk_v)
        lanes = lax.iota(jnp.int32, 16)

        copies = []
        for c in range(CTX):
            cvec = jnp.full((16,), c, jnp.int32)
            for g in range(BPW // 16):
                vals = plsc.load_gather(blk_v, [g * 16 + lanes, cvec])
                idx_v[c, pl.ds(g * 16, 16)] = vals
            copies.append(pltpu.async_copy(
                table_hbm.at[idx_v.at[c]], rows_v.at[c], sem))

        for c in range(CTX):
            copies[c].wait()

            def row_body(i, carry):
                a0, a1 = carry
                return (a0 + rows_v[c, i, pl.ds(0, 16)],
                        a1 + rows_v[c, i, pl.ds(16, 16)])

            z = jnp.zeros((16,), jnp.float32)
            a0, a1 = lax.fori_loop(0, BPW, row_body, (z, z), unroll=8)
            part_v[pl.ds(c * DIM, 16)] = a0
            part_v[pl.ds(c * DIM + 16, 16)] = a1
        pltpu.sync_copy(part_v, out_hbm.at[wid])

    return k(inputs, emb_table)


def _tc_logits(partials, W, b2):
    grid = (pl.cdiv(VOCAB, VT),)

    def body(part_ref, w_ref, b_ref, out_ref, lse_ref, flat_v, m_sc, s_sc):
        i = pl.program_id(0)

        @pl.when(i == 0)
        def _():
            m_sc[0] = -jnp.inf
            s_sc[0] = jnp.float32(0.0)
            flat_v[...] = jnp.sum(part_ref[...], axis=0, keepdims=True)

        logits = lax.dot_general(
            flat_v[...], w_ref[...], (((1,), (1,)), ((), ())),
            preferred_element_type=jnp.float32) + b_ref[...]
        gidx = i * VT + lax.broadcasted_iota(jnp.int32, (1, VT), 1)
        lm = jnp.where(gidx < VOCAB, logits, -jnp.inf)
        m0 = m_sc[0]
        m1 = jnp.maximum(m0, jnp.max(lm))
        s_sc[0] = s_sc[0] * jnp.exp(m0 - m1) + jnp.sum(jnp.exp(lm - m1))
        m_sc[0] = m1
        out_ref[...] = logits

        @pl.when(i == pl.num_programs(0) - 1)
        def _():
            lse_ref[...] = jnp.broadcast_to(m_sc[0] + jnp.log(s_sc[0]), (1, 1))

    return pl.pallas_call(
        body,
        grid=grid,
        in_specs=[
            pl.BlockSpec((NW, CTXDIM), lambda i: (0, 0)),
            pl.BlockSpec((VT, CTXDIM), lambda i: (i, 0)),
            pl.BlockSpec((1, VT), lambda i: (0, i)),
        ],
        out_specs=[
            pl.BlockSpec((1, VT), lambda i: (0, i)),
            pl.BlockSpec((1, 1), lambda i: (0, 0)),
        ],
        out_shape=[
            jax.ShapeDtypeStruct((1, VOCAB), jnp.float32),
            jax.ShapeDtypeStruct((1, 1), jnp.float32),
        ],
        scratch_shapes=[
            pltpu.VMEM((1, CTXDIM), jnp.float32),
            pltpu.SMEM((1,), jnp.float32),
            pltpu.SMEM((1,), jnp.float32),
        ],
    )(partials, W, b2)


def _tc_subtract(logits, lse):
    def body(l_ref, lse_ref, o_ref):
        o_ref[...] = l_ref[...] - lse_ref[0, 0]

    return pl.pallas_call(
        body,
        grid=(pl.cdiv(VOCAB, VTC),),
        in_specs=[
            pl.BlockSpec((1, VTC), lambda i: (0, i)),
            pl.BlockSpec((1, 1), lambda i: (0, 0)),
        ],
        out_specs=pl.BlockSpec((1, VTC), lambda i: (0, i)),
        out_shape=jax.ShapeDtypeStruct((1, VOCAB), jnp.float32),
    )(logits, lse)


def kernel(inputs, emb_table, W, b):
    partials = _sc_gather_sum(inputs, emb_table)
    logits, lse = _tc_logits(partials, W, b.reshape(1, VOCAB))
    return _tc_subtract(logits, lse)


# speedup vs baseline: 1.1357x; 1.0008x over previous
import functools

import jax
import jax.numpy as jnp
from jax import lax
from jax.experimental import pallas as pl
from jax.experimental.pallas import tpu as pltpu
from jax.experimental.pallas import tpu_sc as plsc

VOCAB = 100000
DIM = 32
CTX = 20
BATCH = 4096
CTXDIM = CTX * DIM

NW = 32
BPW = BATCH // NW

VT = 4096
VTC = 12800


def _sc_gather_sum(inputs, emb_table):
    mesh = plsc.VectorSubcoreMesh(core_axis_name="c", subcore_axis_name="s")

    @functools.partial(
        pl.kernel,
        out_type=jax.ShapeDtypeStruct((NW, CTXDIM), jnp.float32),
        mesh=mesh,
        scratch_types=[
            pltpu.VMEM((BPW, CTX), jnp.int32),
            pltpu.VMEM((CTX, BPW), jnp.int32),
            pltpu.VMEM((CTX, BPW, DIM), jnp.float32),
            pltpu.VMEM((CTXDIM,), jnp.float32),
            pltpu.SemaphoreType.DMA,
        ],
        compiler_params=pltpu.CompilerParams(
            use_tc_tiling_on_sc=False, needs_layout_passes=False),
    )
    def k(in_hbm, table_hbm, out_hbm, blk_v, idx_v, rows_v, part_v, sem):
        wid = lax.axis_index("s") * 2 + lax.axis_index("c")
        pltpu.sync_copy(in_hbm.at[pl.ds(wid * BPW, BPW)], blk_v)
        lanes = lax.iota(jnp.int32, 16)

        copies = []
        for c in range(CTX):
            cvec = jnp.full((16,), c, jnp.int32)
            for g in range(BPW // 16):
                vals = plsc.load_gather(blk_v, [g * 16 + lanes, cvec])
                idx_v[c, pl.ds(g * 16, 16)] = vals
            copies.append(pltpu.async_copy(
                table_hbm.at[idx_v.at[c]], rows_v.at[c], sem))

        for c in range(CTX):
            copies[c].wait()

            def row_body(i, carry):
                a0, a1 = carry
                return (a0 + rows_v[c, i, pl.ds(0, 16)],
                        a1 + rows_v[c, i, pl.ds(16, 16)])

            z = jnp.zeros((16,), jnp.float32)
            a0, a1 = lax.fori_loop(0, BPW, row_body, (z, z), unroll=8)
            part_v[pl.ds(c * DIM, 16)] = a0
            part_v[pl.ds(c * DIM + 16, 16)] = a1
        pltpu.sync_copy(part_v, out_hbm.at[wid])

    return k(inputs, emb_table)


def _tc_logits(partials, W, b2):
    grid = (pl.cdiv(VOCAB, VT),)

    def body(part_ref, w_ref, b_ref, out_ref, lse_ref, flat_v, m_sc, s_sc):
        i = pl.program_id(0)

        @pl.when(i == 0)
        def _():
            m_sc[0] = -jnp.inf
            s_sc[0] = jnp.float32(0.0)
            flat_v[...] = jnp.sum(part_ref[...], axis=0, keepdims=True)

        logits = lax.dot_general(
            flat_v[...], w_ref[...], (((1,), (1,)), ((), ())),
            preferred_element_type=jnp.float32) + b_ref[...]
        gidx = i * VT + lax.broadcasted_iota(jnp.int32, (1, VT), 1)
        lm = jnp.where(gidx < VOCAB, logits, -jnp.inf)
        m0 = m_sc[0]
        m1 = jnp.maximum(m0, jnp.max(lm))
        s_sc[0] = s_sc[0] * jnp.exp(m0 - m1) + jnp.sum(jnp.exp(lm - m1))
        m_sc[0] = m1
        out_ref[...] = logits

        @pl.when(i == pl.num_programs(0) - 1)
        def _():
            lse_ref[...] = jnp.broadcast_to(m_sc[0] + jnp.log(s_sc[0]), (1, 1))

    return pl.pallas_call(
        body,
        grid=grid,
        in_specs=[
            pl.BlockSpec((NW, CTXDIM), lambda i: (0, 0)),
            pl.BlockSpec((VT, CTXDIM), lambda i: (i, 0)),
            pl.BlockSpec((1, VT), lambda i: (0, i)),
        ],
        out_specs=[
            pl.BlockSpec((1, VT), lambda i: (0, i)),
            pl.BlockSpec((1, 1), lambda i: (0, 0)),
        ],
        out_shape=[
            jax.ShapeDtypeStruct((1, VOCAB), jnp.float32),
            jax.ShapeDtypeStruct((1, 1), jnp.float32),
        ],
        scratch_shapes=[
            pltpu.VMEM((1, CTXDIM), jnp.float32),
            pltpu.SMEM((1,), jnp.float32),
            pltpu.SMEM((1,), jnp.float32),
        ],
    )(partials, W, b2)


def _tc_subtract(logits, lse):
    def body(l_ref, lse_ref, o_ref):
        o_ref[...] = l_ref[...] - lse_ref[0, 0]

    return pl.pallas_call(
        body,
        grid=(pl.cdiv(VOCAB, VTC),),
        in_specs=[
            pl.BlockSpec((1, VTC), lambda i: (0, i)),
            pl.BlockSpec((1, 1), lambda i: (0, 0)),
        ],
        out_specs=pl.BlockSpec((1, VTC), lambda i: (0, i)),
        out_shape=jax.ShapeDtypeStruct((1, VOCAB), jnp.float32),
    )(logits, lse)


def kernel(inputs, emb_table, W, b):
    partials = _sc_gather_sum(inputs, emb_table)
    logits, lse = _tc_logits(partials, W, b.reshape(1, VOCAB))
    return _tc_subtract(logits, lse)


# speedup vs baseline: 1.1575x; 1.0192x over previous
import functools

import jax
import jax.numpy as jnp
from jax import lax
from jax.experimental import pallas as pl
from jax.experimental.pallas import tpu as pltpu
from jax.experimental.pallas import tpu_sc as plsc

VOCAB = 100000
DIM = 32
CTX = 20
BATCH = 4096
CTXDIM = CTX * DIM

NW = 32
BPW = BATCH // NW

VT = 4096
VTC = VOCAB


def _sc_gather_sum(inputs, emb_table):
    mesh = plsc.VectorSubcoreMesh(core_axis_name="c", subcore_axis_name="s")

    @functools.partial(
        pl.kernel,
        out_type=jax.ShapeDtypeStruct((NW, CTXDIM), jnp.float32),
        mesh=mesh,
        scratch_types=[
            pltpu.VMEM((BPW, CTX), jnp.int32),
            pltpu.VMEM((CTX, BPW), jnp.int32),
            pltpu.VMEM((CTX, BPW, DIM), jnp.float32),
            pltpu.VMEM((CTXDIM,), jnp.float32),
            pltpu.SemaphoreType.DMA,
        ],
        compiler_params=pltpu.CompilerParams(
            use_tc_tiling_on_sc=False, needs_layout_passes=False),
    )
    def k(in_hbm, table_hbm, out_hbm, blk_v, idx_v, rows_v, part_v, sem):
        wid = lax.axis_index("s") * 2 + lax.axis_index("c")
        pltpu.sync_copy(in_hbm.at[pl.ds(wid * BPW, BPW)], blk_v)
        lanes = lax.iota(jnp.int32, 16)

        copies = []
        for c in range(CTX):
            cvec = jnp.full((16,), c, jnp.int32)
            for g in range(BPW // 16):
                vals = plsc.load_gather(blk_v, [g * 16 + lanes, cvec])
                idx_v[c, pl.ds(g * 16, 16)] = vals
            copies.append(pltpu.async_copy(
                table_hbm.at[idx_v.at[c]], rows_v.at[c], sem))

        for c in range(CTX):
            copies[c].wait()

            def row_body(i, carry):
                a0, a1 = carry
                return (a0 + rows_v[c, i, pl.ds(0, 16)],
                        a1 + rows_v[c, i, pl.ds(16, 16)])

            z = jnp.zeros((16,), jnp.float32)
            a0, a1 = lax.fori_loop(0, BPW, row_body, (z, z), unroll=8)
            part_v[pl.ds(c * DIM, 16)] = a0
            part_v[pl.ds(c * DIM + 16, 16)] = a1
        pltpu.sync_copy(part_v, out_hbm.at[wid])

    return k(inputs, emb_table)


def _tc_logits(partials, W, b2):
    grid = (pl.cdiv(VOCAB, VT),)

    def body(part_ref, w_ref, b_ref, out_ref, lse_ref, flat_v, m_sc, s_sc):
        i = pl.program_id(0)

        @pl.when(i == 0)
        def _():
            m_sc[0] = -jnp.inf
            s_sc[0] = jnp.float32(0.0)
            flat_v[...] = jnp.sum(part_ref[...], axis=0, keepdims=True)

        logits = lax.dot_general(
            flat_v[...], w_ref[...], (((1,), (1,)), ((), ())),
            preferred_element_type=jnp.float32) + b_ref[...]
        gidx = i * VT + lax.broadcasted_iota(jnp.int32, (1, VT), 1)
        lm = jnp.where(gidx < VOCAB, logits, -jnp.inf)
        m0 = m_sc[0]
        m1 = jnp.maximum(m0, jnp.max(lm))
        s_sc[0] = s_sc[0] * jnp.exp(m0 - m1) + jnp.sum(jnp.exp(lm - m1))
        m_sc[0] = m1
        out_ref[...] = logits

        @pl.when(i == pl.num_programs(0) - 1)
        def _():
            lse_ref[...] = jnp.broadcast_to(m_sc[0] + jnp.log(s_sc[0]), (1, 1))

    return pl.pallas_call(
        body,
        grid=grid,
        in_specs=[
            pl.BlockSpec((NW, CTXDIM), lambda i: (0, 0)),
            pl.BlockSpec((VT, CTXDIM), lambda i: (i, 0)),
            pl.BlockSpec((1, VT), lambda i: (0, i)),
        ],
        out_specs=[
            pl.BlockSpec((1, VT), lambda i: (0, i)),
            pl.BlockSpec((1, 1), lambda i: (0, 0)),
        ],
        out_shape=[
            jax.ShapeDtypeStruct((1, VOCAB), jnp.float32),
            jax.ShapeDtypeStruct((1, 1), jnp.float32),
        ],
        scratch_shapes=[
            pltpu.VMEM((1, CTXDIM), jnp.float32),
            pltpu.SMEM((1,), jnp.float32),
            pltpu.SMEM((1,), jnp.float32),
        ],
    )(partials, W, b2)


def _tc_subtract(logits, lse):
    def body(l_ref, lse_ref, o_ref):
        o_ref[...] = l_ref[...] - lse_ref[0, 0]

    return pl.pallas_call(
        body,
        grid=(pl.cdiv(VOCAB, VTC),),
        in_specs=[
            pl.BlockSpec((1, VTC), lambda i: (0, i)),
            pl.BlockSpec((1, 1), lambda i: (0, 0)),
        ],
        out_specs=pl.BlockSpec((1, VTC), lambda i: (0, i)),
        out_shape=jax.ShapeDtypeStruct((1, VOCAB), jnp.float32),
    )(logits, lse)


def kernel(inputs, emb_table, W, b):
    partials = _sc_gather_sum(inputs, emb_table)
    logits, lse = _tc_logits(partials, W, b.reshape(1, VOCAB))
    return _tc_subtract(logits, lse)
